# baseline (speedup 1.0000x reference)
import numpy as np

import jax
import jax.numpy as jnp
from jax import lax
from jax.experimental import pallas as pl
from jax.experimental.pallas import tpu as pltpu
from jax.experimental.pallas import tpu_sc as plsc

_ALPHA = 0.2
_LAM = float(np.random.RandomState(0).beta(_ALPHA, _ALPHA))

_NCHUNK = 32
_OBLK = 16

with jax.default_device(jax.devices("cpu")[0]):
    _PERM_NP = np.asarray(
        jax.random.permutation(jax.random.key(42), 64)).astype(np.int32)

_ROWS_PER_W = 2
_LPAD = 528


def _spec_kernel(order_ref, po_ref, needed_ref, x_hbm, ox_ref, buf, sems,
                 waited):
    g = pl.program_id(0)
    nrows = x_hbm.shape[0]
    rpc = nrows // _NCHUNK

    @pl.when(g == 0)
    def _():
        waited[0] = 0
        for c in range(_NCHUNK):
            pltpu.make_async_copy(
                x_hbm.at[pl.ds(c * rpc, rpc)],
                buf.at[pl.ds(c * rpc, rpc)],
                sems.at[c],
            ).start()

    need = needed_ref[g]
    w0 = waited[0]
    for c in range(_NCHUNK):
        @pl.when(jnp.logical_and(c >= w0, c <= need))
        def _(c=c):
            pltpu.make_async_copy(
                x_hbm.at[pl.ds(c * rpc, rpc)],
                buf.at[pl.ds(c * rpc, rpc)],
                sems.at[c],
            ).wait()
    waited[0] = jnp.maximum(w0, need + 1)

    base = order_ref[g] * _OBLK
    for u in range(_OBLK):
        j = po_ref[g * _OBLK + u]
        ox_ref[u, 0] = _LAM * buf[base + u, 0] + (1.0 - _LAM) * buf[j, 0]


def _spec_mix(batch_spectrograms):
    B, C, H, W = batch_spectrograms.shape
    rpc = B // _NCHUNK
    ngrp = B // _OBLK

    rows_np = np.arange(B, dtype=np.int32)
    last_chunk = np.maximum(rows_np // rpc, _PERM_NP // rpc)
    grp_last = last_chunk.reshape(ngrp, _OBLK).max(axis=1)
    order_np = np.argsort(grp_last, kind="stable").astype(np.int32)
    order = jnp.asarray(order_np)
    po_np = _PERM_NP.reshape(ngrp, _OBLK)[order_np].reshape(B)
    po = jnp.asarray(po_np)
    needed = jnp.asarray(grp_last[order_np].astype(np.int32))

    grid_spec = pltpu.PrefetchScalarGridSpec(
        num_scalar_prefetch=3,
        grid=(ngrp,),
        in_specs=[pl.BlockSpec(memory_space=pl.ANY)],
        out_specs=[pl.BlockSpec((_OBLK, C, H, W),
                                lambda g, o, p, n: (o[g], 0, 0, 0))],
        scratch_shapes=[
            pltpu.VMEM((B, C, H, W), jnp.float32),
            pltpu.SemaphoreType.DMA((_NCHUNK,)),
            pltpu.SMEM((1,), jnp.int32),
        ],
    )
    return pl.pallas_call(
        _spec_kernel,
        grid_spec=grid_spec,
        out_shape=[jax.ShapeDtypeStruct(batch_spectrograms.shape, jnp.float32)],
    )(order, po, needed, batch_spectrograms)[0]


def _lab_sc_body(l_hbm, out_hbm, a, b, o, sa, sb, so):
    nc = 2
    wid = lax.axis_index("s") * nc + lax.axis_index("c")

    for k in range(_ROWS_PER_W):
        r = wid * _ROWS_PER_W + k
        q = jnp.int32(_PERM_NP[k])
        for w in range(32):
            q = jnp.where(wid == w, jnp.int32(_PERM_NP[w * _ROWS_PER_W + k]), q)

        ha = pltpu.async_copy(l_hbm.at[r], a, sa)
        hb = pltpu.async_copy(l_hbm.at[q], b, sb)
        ha.wait()
        hb.wait()

        @plsc.parallel_loop(0, _LPAD // 16, unroll=4)
        def _blend(i):
            sl = pl.ds(i * 16, 16)
            o[sl] = _LAM * a[sl] + (1.0 - _LAM) * b[sl]

        pltpu.async_copy(o, out_hbm.at[r], so).wait()


def _lab_mix(batch_labels):
    B, L = batch_labels.shape
    lp = jnp.pad(batch_labels, ((0, 0), (0, _LPAD - L)))
    mesh = plsc.VectorSubcoreMesh(core_axis_name="c", subcore_axis_name="s")
    out = pl.kernel(
        _lab_sc_body,
        mesh=mesh,
        out_type=jax.ShapeDtypeStruct((B, _LPAD), jnp.float32),
        scratch_types=[
            pltpu.VMEM((_LPAD,), jnp.float32),
            pltpu.VMEM((_LPAD,), jnp.float32),
            pltpu.VMEM((_LPAD,), jnp.float32),
            pltpu.SemaphoreType.DMA,
            pltpu.SemaphoreType.DMA,
            pltpu.SemaphoreType.DMA,
        ],
    )(lp)
    return out[:, :L]


def kernel(batch_spectrograms, batch_labels):
    ol = _lab_mix(batch_labels)
    ox = _spec_mix(batch_spectrograms)
    return ox, ol


# speedup vs baseline: 1.0222x; 1.0222x over previous
import numpy as np

import jax
import jax.numpy as jnp
from jax import lax
from jax.experimental import pallas as pl
from jax.experimental.pallas import tpu as pltpu
from jax.experimental.pallas import tpu_sc as plsc

_ALPHA = 0.2
_LAM = float(np.random.RandomState(0).beta(_ALPHA, _ALPHA))

_NCHUNK = 64
_OBLK = 8

with jax.default_device(jax.devices("cpu")[0]):
    _PERM_NP = np.asarray(
        jax.random.permutation(jax.random.key(42), 64)).astype(np.int32)

_ROWS_PER_W = 2
_LPAD = 528


def _spec_kernel(order_ref, po_ref, needed_ref, x_hbm, ox_ref, buf, sems,
                 waited):
    g = pl.program_id(0)
    nrows = x_hbm.shape[0]
    rpc = nrows // _NCHUNK

    @pl.when(g == 0)
    def _():
        waited[0] = 0
        for c in range(_NCHUNK):
            pltpu.make_async_copy(
                x_hbm.at[pl.ds(c * rpc, rpc)],
                buf.at[pl.ds(c * rpc, rpc)],
                sems.at[c],
            ).start()

    need = needed_ref[g]
    w0 = waited[0]
    for c in range(_NCHUNK):
        @pl.when(jnp.logical_and(c >= w0, c <= need))
        def _(c=c):
            pltpu.make_async_copy(
                x_hbm.at[pl.ds(c * rpc, rpc)],
                buf.at[pl.ds(c * rpc, rpc)],
                sems.at[c],
            ).wait()
    waited[0] = jnp.maximum(w0, need + 1)

    base = order_ref[g] * _OBLK
    for u in range(_OBLK):
        j = po_ref[g * _OBLK + u]
        ox_ref[u, 0] = _LAM * buf[base + u, 0] + (1.0 - _LAM) * buf[j, 0]


def _spec_mix(batch_spectrograms):
    B, C, H, W = batch_spectrograms.shape
    rpc = B // _NCHUNK
    ngrp = B // _OBLK

    rows_np = np.arange(B, dtype=np.int32)
    last_chunk = np.maximum(rows_np // rpc, _PERM_NP // rpc)
    grp_last = last_chunk.reshape(ngrp, _OBLK).max(axis=1)
    order_np = np.argsort(grp_last, kind="stable").astype(np.int32)
    order = jnp.asarray(order_np)
    po_np = _PERM_NP.reshape(ngrp, _OBLK)[order_np].reshape(B)
    po = jnp.asarray(po_np)
    needed = jnp.asarray(grp_last[order_np].astype(np.int32))

    grid_spec = pltpu.PrefetchScalarGridSpec(
        num_scalar_prefetch=3,
        grid=(ngrp,),
        in_specs=[pl.BlockSpec(memory_space=pl.ANY)],
        out_specs=[pl.BlockSpec((_OBLK, C, H, W),
                                lambda g, o, p, n: (o[g], 0, 0, 0))],
        scratch_shapes=[
            pltpu.VMEM((B, C, H, W), jnp.float32),
            pltpu.SemaphoreType.DMA((_NCHUNK,)),
            pltpu.SMEM((1,), jnp.int32),
        ],
    )
    return pl.pallas_call(
        _spec_kernel,
        grid_spec=grid_spec,
        out_shape=[jax.ShapeDtypeStruct(batch_spectrograms.shape, jnp.float32)],
    )(order, po, needed, batch_spectrograms)[0]


def _lab_sc_body(l_hbm, out_hbm, a, b, o, sa, sb, so):
    nc = 2
    wid = lax.axis_index("s") * nc + lax.axis_index("c")

    for k in range(_ROWS_PER_W):
        r = wid * _ROWS_PER_W + k
        q = jnp.int32(_PERM_NP[k])
        for w in range(32):
            q = jnp.where(wid == w, jnp.int32(_PERM_NP[w * _ROWS_PER_W + k]), q)

        ha = pltpu.async_copy(l_hbm.at[r], a, sa)
        hb = pltpu.async_copy(l_hbm.at[q], b, sb)
        ha.wait()
        hb.wait()

        @plsc.parallel_loop(0, _LPAD // 16, unroll=4)
        def _blend(i):
            sl = pl.ds(i * 16, 16)
            o[sl] = _LAM * a[sl] + (1.0 - _LAM) * b[sl]

        pltpu.async_copy(o, out_hbm.at[r], so).wait()


def _lab_mix(batch_labels):
    B, L = batch_labels.shape
    lp = jnp.pad(batch_labels, ((0, 0), (0, _LPAD - L)))
    mesh = plsc.VectorSubcoreMesh(core_axis_name="c", subcore_axis_name="s")
    out = pl.kernel(
        _lab_sc_body,
        mesh=mesh,
        out_type=jax.ShapeDtypeStruct((B, _LPAD), jnp.float32),
        scratch_types=[
            pltpu.VMEM((_LPAD,), jnp.float32),
            pltpu.VMEM((_LPAD,), jnp.float32),
            pltpu.VMEM((_LPAD,), jnp.float32),
            pltpu.SemaphoreType.DMA,
            pltpu.SemaphoreType.DMA,
            pltpu.SemaphoreType.DMA,
        ],
    )(lp)
    return out[:, :L]


def kernel(batch_spectrograms, batch_labels):
    ol = _lab_mix(batch_labels)
    ox = _spec_mix(batch_spectrograms)
    return ox, ol


# speedup vs baseline: 1.0335x; 1.0111x over previous
import numpy as np

import jax
import jax.numpy as jnp
from jax import lax
from jax.experimental import pallas as pl
from jax.experimental.pallas import tpu as pltpu
from jax.experimental.pallas import tpu_sc as plsc

_ALPHA = 0.2
_LAM = float(np.random.RandomState(0).beta(_ALPHA, _ALPHA))

_NCHUNK = 32
_OBLK = 8

with jax.default_device(jax.devices("cpu")[0]):
    _PERM_NP = np.asarray(
        jax.random.permutation(jax.random.key(42), 64)).astype(np.int32)

_ROWS_PER_W = 2
_LPAD = 528


def _spec_kernel(order_ref, po_ref, needed_ref, x_hbm, ox_ref, buf, sems,
                 waited):
    g = pl.program_id(0)
    nrows = x_hbm.shape[0]
    rpc = nrows // _NCHUNK

    @pl.when(g == 0)
    def _():
        waited[0] = 0
        for c in range(_NCHUNK):
            pltpu.make_async_copy(
                x_hbm.at[pl.ds(c * rpc, rpc)],
                buf.at[pl.ds(c * rpc, rpc)],
                sems.at[c],
            ).start()

    need = needed_ref[g]
    w0 = waited[0]
    for c in range(_NCHUNK):
        @pl.when(jnp.logical_and(c >= w0, c <= need))
        def _(c=c):
            pltpu.make_async_copy(
                x_hbm.at[pl.ds(c * rpc, rpc)],
                buf.at[pl.ds(c * rpc, rpc)],
                sems.at[c],
            ).wait()
    waited[0] = jnp.maximum(w0, need + 1)

    base = order_ref[g] * _OBLK
    for u in range(_OBLK):
        j = po_ref[g * _OBLK + u]
        ox_ref[u, 0] = _LAM * buf[base + u, 0] + (1.0 - _LAM) * buf[j, 0]


def _spec_mix(batch_spectrograms):
    B, C, H, W = batch_spectrograms.shape
    rpc = B // _NCHUNK
    ngrp = B // _OBLK

    rows_np = np.arange(B, dtype=np.int32)
    last_chunk = np.maximum(rows_np // rpc, _PERM_NP // rpc)
    grp_last = last_chunk.reshape(ngrp, _OBLK).max(axis=1)
    order_np = np.argsort(grp_last, kind="stable").astype(np.int32)
    order = jnp.asarray(order_np)
    po_np = _PERM_NP.reshape(ngrp, _OBLK)[order_np].reshape(B)
    po = jnp.asarray(po_np)
    needed = jnp.asarray(grp_last[order_np].astype(np.int32))

    grid_spec = pltpu.PrefetchScalarGridSpec(
        num_scalar_prefetch=3,
        grid=(ngrp,),
        in_specs=[pl.BlockSpec(memory_space=pl.ANY)],
        out_specs=[pl.BlockSpec((_OBLK, C, H, W),
                                lambda g, o, p, n: (o[g], 0, 0, 0))],
        scratch_shapes=[
            pltpu.VMEM((B, C, H, W), jnp.float32),
            pltpu.SemaphoreType.DMA((_NCHUNK,)),
            pltpu.SMEM((1,), jnp.int32),
        ],
    )
    return pl.pallas_call(
        _spec_kernel,
        grid_spec=grid_spec,
        out_shape=[jax.ShapeDtypeStruct(batch_spectrograms.shape, jnp.float32)],
    )(order, po, needed, batch_spectrograms)[0]


def _lab_sc_body(l_hbm, out_hbm, a, b, o, sa, sb, so):
    nc = 2
    wid = lax.axis_index("s") * nc + lax.axis_index("c")

    for k in range(_ROWS_PER_W):
        r = wid * _ROWS_PER_W + k
        q = jnp.int32(_PERM_NP[k])
        for w in range(32):
            q = jnp.where(wid == w, jnp.int32(_PERM_NP[w * _ROWS_PER_W + k]), q)

        ha = pltpu.async_copy(l_hbm.at[r], a, sa)
        hb = pltpu.async_copy(l_hbm.at[q], b, sb)
        ha.wait()
        hb.wait()

        @plsc.parallel_loop(0, _LPAD // 16, unroll=4)
        def _blend(i):
            sl = pl.ds(i * 16, 16)
            o[sl] = _LAM * a[sl] + (1.0 - _LAM) * b[sl]

        pltpu.async_copy(o, out_hbm.at[r], so).wait()


def _lab_mix(batch_labels):
    B, L = batch_labels.shape
    lp = jnp.pad(batch_labels, ((0, 0), (0, _LPAD - L)))
    mesh = plsc.VectorSubcoreMesh(core_axis_name="c", subcore_axis_name="s")
    out = pl.kernel(
        _lab_sc_body,
        mesh=mesh,
        out_type=jax.ShapeDtypeStruct((B, _LPAD), jnp.float32),
        scratch_types=[
            pltpu.VMEM((_LPAD,), jnp.float32),
            pltpu.VMEM((_LPAD,), jnp.float32),
            pltpu.VMEM((_LPAD,), jnp.float32),
            pltpu.SemaphoreType.DMA,
            pltpu.SemaphoreType.DMA,
            pltpu.SemaphoreType.DMA,
        ],
    )(lp)
    return out[:, :L]


def kernel(batch_spectrograms, batch_labels):
    ol = _lab_mix(batch_labels)
    ox = _spec_mix(batch_spectrograms)
    return ox, ol
